# Initial kernel scaffold; baseline (speedup 1.0000x reference)
import jax
import jax.numpy as jnp
from jax.experimental import pallas as pl


def kernel(xi, xj, edge_index, params):
    raise NotImplementedError("write your pallas kernel here")


# speedup vs baseline: 1.0463x; 1.0463x over previous
import functools

import jax
import jax.numpy as jnp
from jax.experimental import pallas as pl

N = 10000
E = 320000
D = 128


def _matmul_body(x_ref, w_ref, b_ref, o_ref, *, act):
    x = x_ref[...]
    if act == "elu":
        x = jnp.where(x > 0, x, jnp.exp(jnp.minimum(x, 0.0)) - 1.0)
    o_ref[...] = jnp.dot(x, w_ref[...], preferred_element_type=jnp.float32) + b_ref[...]


def _proj(x, w, b, act="none", block_m=400):
    m, kdim = x.shape
    kdim2, n = w.shape
    grid = (m // block_m,)
    return pl.pallas_call(
        functools.partial(_matmul_body, act=act),
        grid=grid,
        in_specs=[
            pl.BlockSpec((block_m, kdim), lambda i: (i, 0)),
            pl.BlockSpec((kdim, n), lambda i: (0, 0)),
            pl.BlockSpec((1, n), lambda i: (0, 0)),
        ],
        out_specs=pl.BlockSpec((block_m, n), lambda i: (i, 0)),
        out_shape=jax.ShapeDtypeStruct((m, n), jnp.float32),
    )(x, w, b.reshape(1, n))


def _attn_edges(q, k, v, src, dst):
    dout = q.shape[-1]
    score = jnp.sum(q[dst] * k[src], axis=-1) / jnp.sqrt(jnp.float32(dout))
    mx = jax.ops.segment_max(score, dst, num_segments=N)
    mx = jnp.where(jnp.isfinite(mx), mx, 0.0)
    ex = jnp.exp(score - mx[dst])
    denom = jax.ops.segment_sum(ex, dst, num_segments=N)
    alpha = ex / (denom[dst] + 1e-16)
    return jax.ops.segment_sum(alpha[:, None] * v[src], dst, num_segments=N)


def _conv(x, src, dst, p, act="none"):
    wcat = jnp.concatenate([p["Wq"], p["Wk"], p["Wv"], p["Ws"]], axis=1)
    bcat = jnp.concatenate([p["bq"], p["bk"], p["bv"], p["bs"]])
    dout = p["Wq"].shape[1]
    qkvs = _proj(x, wcat, bcat, act=act)
    q, k, v, s = (qkvs[:, i * dout:(i + 1) * dout] for i in range(4))
    return _attn_edges(q, k, v, src, dst) + s


def kernel(xi, xj, edge_index, params):
    src = edge_index[0]
    dst = edge_index[1]
    h1i = _conv(xi, src, dst, params["conv1"])
    hi2 = _conv(h1i, src, dst, params["conv2"], act="elu")
    ci = _conv(h1i, src, dst, params["proj"], act="elu")
    h1j = _conv(xj, src, dst, params["conv1"])
    hj2 = _conv(h1j, src, dst, params["conv2"], act="elu")
    cj = _conv(hj2, src, dst, params["proj"])
    return (hi2, hj2, ci, cj)
